# baseline (speedup 1.0000x reference)
import functools

import jax
import jax.numpy as jnp
from jax import lax
from jax.experimental import pallas as pl
from jax.experimental.pallas import tpu as pltpu
from jax.experimental.pallas import tpu_sc as plsc

N_NODES = 10000
N_EDGES = 160000
D_FEAT = 256

NUM_CORES = 2
NUM_TILES = 16
HALF = D_FEAT // NUM_CORES
EDGES_PER_TILE = N_EDGES // NUM_TILES
CHUNK = 128
NCH = 78
TAIL = EDGES_PER_TILE - NCH * CHUNK
NBUF = 3
IBUF = 4
ROWS_PER_TILE = N_NODES // NUM_TILES
WB_ROWS = 624
WB_LAST_BASE = (NUM_TILES - 1) * WB_ROWS
WB_LAST = N_NODES - WB_LAST_BASE

_mesh = plsc.VectorSubcoreMesh(core_axis_name="c", subcore_axis_name="s")


@functools.partial(
    pl.kernel,
    out_type=jax.ShapeDtypeStruct((N_NODES, D_FEAT), jnp.float32),
    mesh=_mesh,
    scratch_types=[
        pltpu.VMEM((IBUF, CHUNK), jnp.int32),
        pltpu.VMEM((IBUF, CHUNK), jnp.int32),
        pltpu.VMEM((NBUF, CHUNK, HALF), jnp.float32),
        pltpu.VMEM_SHARED((N_NODES, HALF), jnp.float32),
        [pltpu.SemaphoreType.DMA] * IBUF,
        [pltpu.SemaphoreType.DMA] * IBUF,
        [pltpu.SemaphoreType.DMA] * NBUF,
        [pltpu.SemaphoreType.DMA] * NBUF,
    ],
)
def _mp_kernel(x, src_hbm, dst_hbm, out,
               src_ring, dst_ring, rows, acc,
               sem_si, sem_di, sem_g, sem_s):
    cid = lax.axis_index("c")
    sid = lax.axis_index("s")
    base0 = sid * EDGES_PER_TILE
    col0 = pl.multiple_of(cid * HALF, HALF)

    zeros16 = jnp.zeros((16,), jnp.float32)

    @pl.loop(0, CHUNK)
    def _zero(r):
        @pl.loop(0, HALF // 16)
        def _zrow(c):
            rows[0, r, pl.ds(c * 16, 16)] = zeros16

    @pl.loop(0, ROWS_PER_TILE // CHUNK)
    def _zacc(j):
        pltpu.sync_copy(
            rows.at[0], acc.at[pl.ds(sid * ROWS_PER_TILE + j * CHUNK, CHUNK)])

    _ztail = ROWS_PER_TILE - (ROWS_PER_TILE // CHUNK) * CHUNK
    pltpu.sync_copy(
        rows.at[0, pl.ds(0, _ztail)],
        acc.at[pl.ds(sid * ROWS_PER_TILE + ROWS_PER_TILE - _ztail, _ztail)])

    plsc.subcore_barrier()

    def hbm_idx(arr, c, n=CHUNK):
        return arr.at[pl.ds(pl.multiple_of(base0 + c * CHUNK, 8), n)]

    def start_src_idx(c, b):
        pltpu.async_copy(hbm_idx(src_hbm, c), src_ring.at[b], sem_si[b])

    def wait_src_idx(c, b):
        pltpu.make_async_copy(
            hbm_idx(src_hbm, c), src_ring.at[b], sem_si[b]).wait()

    def start_dst_idx(c, b):
        pltpu.async_copy(hbm_idx(dst_hbm, c), dst_ring.at[b], sem_di[b])

    def wait_dst_idx(c, b):
        pltpu.make_async_copy(
            hbm_idx(dst_hbm, c), dst_ring.at[b], sem_di[b]).wait()

    def start_gather(c, b, bi):
        pltpu.async_copy(
            x.at[src_ring.at[bi], pl.ds(col0, HALF)], rows.at[b], sem_g[b])

    def wait_gather(c, b, bi):
        pltpu.make_async_copy(
            x.at[src_ring.at[bi], pl.ds(col0, HALF)], rows.at[b],
            sem_g[b]).wait()

    def start_scatter(c, b, bi):
        pltpu.async_copy(
            rows.at[b], acc.at[dst_ring.at[bi]], sem_s[b], add=True)

    def wait_scatter(c, b, bi):
        pltpu.make_async_copy(
            rows.at[b], acc.at[dst_ring.at[bi]], sem_s[b]).wait()

    for k in range(NBUF):
        start_src_idx(k, k)
        start_dst_idx(k, k)
    for k in range(2):
        wait_src_idx(k, k)
        start_gather(k, k, k)

    @pl.loop(0, NCH + 11 - (NCH + 11) % 12, step=12)
    def _main(i):
        for b12 in range(12):
            c = i + b12
            b = b12 % NBUF
            bi = b12 % IBUF
            b2 = (b12 + 2) % NBUF
            bi2 = (b12 + 2) % IBUF
            bi3 = (b12 + 3) % IBUF

            @pl.when(c < NCH)
            def _():
                wait_gather(c, b, bi)

                @pl.when(c >= 1)
                def _():
                    wait_scatter(c - 1, b2, bi3)

                @pl.when(c + 3 < NCH)
                def _():
                    start_src_idx(c + 3, bi3)
                    start_dst_idx(c + 3, bi3)

                @pl.when(c + 2 < NCH)
                def _():
                    wait_src_idx(c + 2, bi2)
                    start_gather(c + 2, b2, bi2)

                wait_dst_idx(c, bi)
                start_scatter(c, b, bi)

    wait_scatter(NCH - 1, (NCH - 1) % NBUF, (NCH - 1) % IBUF)

    pltpu.sync_copy(hbm_idx(src_hbm, NCH, TAIL), src_ring.at[0, pl.ds(0, TAIL)])
    pltpu.sync_copy(hbm_idx(dst_hbm, NCH, TAIL), dst_ring.at[0, pl.ds(0, TAIL)])
    pltpu.async_copy(
        x.at[src_ring.at[0, pl.ds(0, TAIL)], pl.ds(col0, HALF)],
        rows.at[0, pl.ds(0, TAIL)], sem_g[0]).wait()
    pltpu.sync_copy(
        rows.at[0, pl.ds(0, TAIL)],
        acc.at[dst_ring.at[0, pl.ds(0, TAIL)]], add=True)

    plsc.subcore_barrier()

    @pl.when(sid < NUM_TILES - 1)
    def _wb():
        base = pl.multiple_of(sid * WB_ROWS, 8)
        pltpu.sync_copy(
            acc.at[pl.ds(base, WB_ROWS)],
            out.at[pl.ds(base, WB_ROWS), pl.ds(col0, HALF)],
        )

    @pl.when(sid == NUM_TILES - 1)
    def _wb_last():
        pltpu.sync_copy(
            acc.at[pl.ds(WB_LAST_BASE, WB_LAST)],
            out.at[pl.ds(WB_LAST_BASE, WB_LAST), pl.ds(col0, HALF)],
        )


def kernel(x, edge_index):
    ei = edge_index.astype(jnp.int32)
    return _mp_kernel(x, ei[0], ei[1])


# speedup vs baseline: 1.0021x; 1.0021x over previous
import functools

import jax
import jax.numpy as jnp
from jax import lax
from jax.experimental import pallas as pl
from jax.experimental.pallas import tpu as pltpu
from jax.experimental.pallas import tpu_sc as plsc

N_NODES = 10000
N_EDGES = 160000
D_FEAT = 256

NUM_CORES = 2
NUM_TILES = 16
HALF = D_FEAT // NUM_CORES
EDGES_PER_TILE = N_EDGES // NUM_TILES
CHUNK = 80
NUM_CHUNKS = EDGES_PER_TILE // CHUNK
NBUF = 4
ROWS_PER_TILE = N_NODES // NUM_TILES
WB_ROWS = 624
WB_LAST_BASE = (NUM_TILES - 1) * WB_ROWS
WB_LAST = N_NODES - WB_LAST_BASE

_mesh = plsc.VectorSubcoreMesh(core_axis_name="c", subcore_axis_name="s")


@functools.partial(
    pl.kernel,
    out_type=jax.ShapeDtypeStruct((N_NODES, D_FEAT), jnp.float32),
    mesh=_mesh,
    scratch_types=[
        pltpu.VMEM((NBUF, CHUNK), jnp.int32),
        pltpu.VMEM((NBUF, CHUNK), jnp.int32),
        pltpu.VMEM((NBUF, CHUNK, HALF), jnp.float32),
        pltpu.VMEM_SHARED((N_NODES, HALF), jnp.float32),
        [pltpu.SemaphoreType.DMA] * NBUF,
        [pltpu.SemaphoreType.DMA] * NBUF,
        [pltpu.SemaphoreType.DMA] * NBUF,
        [pltpu.SemaphoreType.DMA] * NBUF,
    ],
)
def _mp_kernel(x, src_hbm, dst_hbm, out,
               src_ring, dst_ring, rows, acc,
               sem_si, sem_di, sem_g, sem_s):
    cid = lax.axis_index("c")
    sid = lax.axis_index("s")
    base0 = sid * EDGES_PER_TILE
    col0 = pl.multiple_of(cid * HALF, HALF)

    zeros16 = jnp.zeros((16,), jnp.float32)

    @pl.loop(0, CHUNK)
    def _zero(r):
        @pl.loop(0, HALF // 16)
        def _zrow(c):
            rows[0, r, pl.ds(c * 16, 16)] = zeros16

    @pl.loop(0, ROWS_PER_TILE // CHUNK)
    def _zacc(j):
        pltpu.sync_copy(
            rows.at[0], acc.at[pl.ds(sid * ROWS_PER_TILE + j * CHUNK, CHUNK)])

    _ztail = ROWS_PER_TILE - (ROWS_PER_TILE // CHUNK) * CHUNK
    pltpu.sync_copy(
        rows.at[0, pl.ds(0, _ztail)],
        acc.at[pl.ds(sid * ROWS_PER_TILE + ROWS_PER_TILE - _ztail, _ztail)])

    plsc.subcore_barrier()

    def hbm_idx(arr, c):
        return arr.at[pl.ds(pl.multiple_of(base0 + c * CHUNK, 8), CHUNK)]

    def start_src_idx(c, b):
        pltpu.async_copy(hbm_idx(src_hbm, c), src_ring.at[b], sem_si[b])

    def wait_src_idx(c, b):
        pltpu.make_async_copy(
            hbm_idx(src_hbm, c), src_ring.at[b], sem_si[b]).wait()

    def start_dst_idx(c, b):
        pltpu.async_copy(hbm_idx(dst_hbm, c), dst_ring.at[b], sem_di[b])

    def wait_dst_idx(c, b):
        pltpu.make_async_copy(
            hbm_idx(dst_hbm, c), dst_ring.at[b], sem_di[b]).wait()

    def start_gather(c, b):
        pltpu.async_copy(
            x.at[src_ring.at[b], pl.ds(col0, HALF)], rows.at[b], sem_g[b])

    def wait_gather(c, b):
        pltpu.make_async_copy(
            x.at[src_ring.at[b], pl.ds(col0, HALF)], rows.at[b],
            sem_g[b]).wait()

    def start_scatter(c, b):
        pltpu.async_copy(
            rows.at[b], acc.at[dst_ring.at[b]], sem_s[b], add=True)

    def wait_scatter(c, b):
        pltpu.make_async_copy(
            rows.at[b], acc.at[dst_ring.at[b]], sem_s[b]).wait()

    for k in range(NBUF):
        start_src_idx(k, k)
    for k in range(2):
        start_dst_idx(k, k)
    for k in range(2):
        wait_src_idx(k, k)
        start_gather(k, k)

    @pl.loop(0, NUM_CHUNKS, step=NBUF)
    def _step(i):
        for b in range(NBUF):
            c = i + b
            b2 = (b + 2) % NBUF

            @pl.when(c < NUM_CHUNKS)
            def _():
                wait_gather(c, b)

                @pl.when(c + NBUF < NUM_CHUNKS)
                def _():
                    start_src_idx(c + NBUF, b)

                @pl.when(c >= 2)
                def _():
                    wait_scatter(c - 2, b2)

                @pl.when(c + 2 < NUM_CHUNKS)
                def _():
                    start_dst_idx(c + 2, b2)
                    wait_src_idx(c + 2, b2)
                    start_gather(c + 2, b2)

                wait_dst_idx(c, b)
                start_scatter(c, b)

    for k in range(min(2, NUM_CHUNKS)):
        cc = NUM_CHUNKS - 1 - k
        wait_scatter(cc, cc % NBUF)

    plsc.subcore_barrier()

    @pl.when(sid < NUM_TILES - 1)
    def _wb():
        base = pl.multiple_of(sid * WB_ROWS, 8)
        pltpu.sync_copy(
            acc.at[pl.ds(base, WB_ROWS)],
            out.at[pl.ds(base, WB_ROWS), pl.ds(col0, HALF)],
        )

    @pl.when(sid == NUM_TILES - 1)
    def _wb_last():
        pltpu.sync_copy(
            acc.at[pl.ds(WB_LAST_BASE, WB_LAST)],
            out.at[pl.ds(WB_LAST_BASE, WB_LAST), pl.ds(col0, HALF)],
        )


def kernel(x, edge_index):
    ei = edge_index.astype(jnp.int32)
    return _mp_kernel(x, ei[0], ei[1])
